# Initial kernel scaffold; baseline (speedup 1.0000x reference)
import jax
import jax.numpy as jnp
from jax.experimental import pallas as pl


def kernel(sparse_repr, input_ids, attention_mask):
    raise NotImplementedError("write your pallas kernel here")


# speedup vs baseline: 1.0805x; 1.0805x over previous
import functools

import jax
import jax.numpy as jnp
from jax import lax
from jax.experimental import pallas as pl
from jax.experimental.pallas import tpu as pltpu
from jax.experimental.pallas import tpu_sc as plsc

_NC = 2
_NS = 16
_NW = _NC * _NS
_CHUNK = 128


def _sc_gather_body(n_chunks, x_hbm, idx_hbm, out_hbm, idx_v, row_v, sem):
    wid = lax.axis_index("s") * _NC + lax.axis_index("c")
    pltpu.sync_copy(idx_hbm.at[wid], idx_v)

    def chunk(j, carry):
        pltpu.async_copy(x_hbm.at[idx_v.at[j]], row_v, sem).wait()
        pltpu.sync_copy(row_v, out_hbm.at[wid, j])
        return carry

    lax.fori_loop(0, n_chunks, chunk, 0, unroll=False)


def _sc_gather(x_flat, idx3):
    n_chunks = idx3.shape[1]
    mesh = plsc.VectorSubcoreMesh(
        core_axis_name="c", subcore_axis_name="s", num_cores=_NC,
        num_subcores=_NS)
    kern = pl.kernel(
        functools.partial(_sc_gather_body, n_chunks),
        out_type=jax.ShapeDtypeStruct((_NW, n_chunks, _CHUNK), jnp.float32),
        mesh=mesh,
        scratch_types=[
            pltpu.VMEM((n_chunks, _CHUNK), jnp.int32),
            pltpu.VMEM((_CHUNK,), jnp.float32),
            pltpu.SemaphoreType.DMA,
        ],
    )
    return kern(x_flat, idx3)


def _tc_body(x_ref, ids_ref, m_ref, vals_ref, o_ref):
    x = x_ref[...]
    dense = jnp.sum(jnp.maximum(x, 0.0) + jnp.log(1.0 + jnp.exp(-jnp.abs(x))))

    ids = ids_ref[...]
    m = m_ref[...]
    vals = vals_ref[...]
    lp = ids.shape[1]
    eq = ids[:, :, None] == ids[:, None, :]
    ii = lax.broadcasted_iota(jnp.int32, (1, lp, lp), 1)
    jj = lax.broadcasted_iota(jnp.int32, (1, lp, lp), 2)
    tri = jj < ii
    contrib = jnp.where(eq & tri, m[:, None, :], 0.0)
    p_prev = jnp.sum(contrib, axis=2)
    w = jnp.minimum(p_prev + m, 1.0) - jnp.minimum(p_prev, 1.0)
    corr = jnp.sum(w * vals)

    @pl.when(pl.program_id(0) == 0)
    def _():
        o_ref[...] = jnp.zeros_like(o_ref)

    o_ref[...] += jnp.full((1, 1), dense - corr, jnp.float32)


def kernel(sparse_repr, input_ids, attention_mask):
    b, v = sparse_repr.shape
    l = input_ids.shape[1]
    ids = input_ids.astype(jnp.int32)
    mask = attention_mask.astype(jnp.float32)

    flat_idx = (ids + jnp.arange(b, dtype=jnp.int32)[:, None] * v).reshape(-1)
    assert (b * l) % (_NW * _CHUNK) == 0
    idx3 = flat_idx.reshape(_NW, -1, _CHUNK)
    vals = _sc_gather(sparse_repr.reshape(b * v), idx3).reshape(b, l)

    lp = (l + 127) // 128 * 128
    ids_p = jnp.pad(ids, ((0, 0), (0, lp - l)), constant_values=-1)
    m_p = jnp.pad(mask, ((0, 0), (0, lp - l)))
    vals_p = jnp.pad(vals, ((0, 0), (0, lp - l)))

    bb = 8
    grid = (b // bb,)
    tot = pl.pallas_call(
        _tc_body,
        grid=grid,
        in_specs=[
            pl.BlockSpec((bb, v), lambda i: (i, 0)),
            pl.BlockSpec((bb, lp), lambda i: (i, 0)),
            pl.BlockSpec((bb, lp), lambda i: (i, 0)),
            pl.BlockSpec((bb, lp), lambda i: (i, 0)),
        ],
        out_specs=pl.BlockSpec((1, 1), lambda i: (0, 0)),
        out_shape=jax.ShapeDtypeStruct((1, 1), jnp.float32),
    )(sparse_repr, ids_p, m_p, vals_p)

    return tot[0, 0] / (b * v)


# speedup vs baseline: 1.8230x; 1.6871x over previous
import functools

import jax
import jax.numpy as jnp
from jax import lax
from jax.experimental import pallas as pl
from jax.experimental.pallas import tpu as pltpu
from jax.experimental.pallas import tpu_sc as plsc

_NC = 2
_NS = 16
_NW = _NC * _NS
_CHUNK = 128


def _sc_gather_body(n_chunks, x_hbm, idx_hbm, out_hbm, idx_v, rows_v, sem):
    wid = lax.axis_index("s") * _NC + lax.axis_index("c")
    pltpu.sync_copy(idx_hbm.at[wid], idx_v)

    def issue(j, carry):
        pltpu.async_copy(
            x_hbm.at[idx_v.at[j]], rows_v.at[pl.ds(j * _CHUNK, _CHUNK)], sem)
        return carry

    lax.fori_loop(0, n_chunks, issue, 0, unroll=False)
    pltpu.make_async_copy(
        x_hbm.at[pl.ds(0, n_chunks * _CHUNK)], rows_v, sem).wait()
    pltpu.sync_copy(rows_v, out_hbm.at[wid])


def _sc_gather(x_flat, idx3):
    n_chunks = idx3.shape[1]
    mesh = plsc.VectorSubcoreMesh(
        core_axis_name="c", subcore_axis_name="s", num_cores=_NC,
        num_subcores=_NS)
    kern = pl.kernel(
        functools.partial(_sc_gather_body, n_chunks),
        out_type=jax.ShapeDtypeStruct((_NW, n_chunks * _CHUNK), jnp.float32),
        mesh=mesh,
        scratch_types=[
            pltpu.VMEM((n_chunks, _CHUNK), jnp.int32),
            pltpu.VMEM((n_chunks * _CHUNK,), jnp.float32),
            pltpu.SemaphoreType.DMA,
        ],
    )
    return kern(x_flat, idx3)


def _dense_body(bb, v, vp, x_ref, o_ref, flat_ref, xp_ref, sem):
    i = pl.program_id(0)
    x = x_ref[...]
    xp_ref[:, :v] = x
    copies = [
        pltpu.make_async_copy(
            xp_ref.at[r], flat_ref.at[pl.ds((i * bb + r) * vp, vp)], sem)
        for r in range(bb)
    ]
    for c in copies:
        c.start()

    dense = jnp.sum(jnp.maximum(x, 0.0) + jnp.log(1.0 + jnp.exp(-jnp.abs(x))))

    @pl.when(i == 0)
    def _():
        o_ref[...] = jnp.zeros_like(o_ref)

    o_ref[...] += jnp.full((1, 1), dense, jnp.float32)
    for c in copies:
        c.wait()


def _corr_body(l_real, ids_ref, m_ref, vals_ref, o_ref):
    ids = ids_ref[...]
    m = m_ref[...]
    vals = vals_ref[...]
    lu = (l_real + 7) // 8 * 8
    eq = ids[:, :lu, None] == ids[:, None, :]
    mt = jnp.sum(jnp.where(eq, m[:, :lu, None], 0.0), axis=1)
    w = jnp.where(mt != 0.0, m * jnp.minimum(mt, 1.0) / mt, 0.0)
    corr = jnp.sum(w * vals)

    @pl.when(pl.program_id(0) == 0)
    def _():
        o_ref[...] = jnp.zeros_like(o_ref)

    o_ref[...] += jnp.full((1, 1), corr, jnp.float32)


def kernel(sparse_repr, input_ids, attention_mask):
    b, v = sparse_repr.shape
    l = input_ids.shape[1]
    ids = input_ids.astype(jnp.int32)
    mask = attention_mask.astype(jnp.float32)

    bb = 8
    vp = (v + 127) // 128 * 128
    dense_tot, x_flat = pl.pallas_call(
        functools.partial(_dense_body, bb, v, vp),
        grid=(b // bb,),
        in_specs=[pl.BlockSpec((bb, v), lambda i: (i, 0))],
        out_specs=[
            pl.BlockSpec((1, 1), lambda i: (0, 0)),
            pl.BlockSpec(memory_space=pl.ANY),
        ],
        out_shape=[
            jax.ShapeDtypeStruct((1, 1), jnp.float32),
            jax.ShapeDtypeStruct((b * vp,), jnp.float32),
        ],
        scratch_shapes=[
            pltpu.VMEM((bb, vp), jnp.float32),
            pltpu.SemaphoreType.DMA,
        ],
    )(sparse_repr)

    flat_idx = (ids + jnp.arange(b, dtype=jnp.int32)[:, None] * vp).reshape(-1)
    assert (b * l) % (_NW * _CHUNK) == 0
    idx3 = flat_idx.reshape(_NW, -1, _CHUNK)
    vals = _sc_gather(x_flat, idx3).reshape(b, l)

    lp = (l + 127) // 128 * 128
    ids_p = jnp.pad(ids, ((0, 0), (0, lp - l)), constant_values=-1)
    m_p = jnp.pad(mask, ((0, 0), (0, lp - l)))
    vals_p = jnp.pad(vals, ((0, 0), (0, lp - l)))

    bc = 16
    corr_tot = pl.pallas_call(
        functools.partial(_corr_body, l),
        grid=(b // bc,),
        in_specs=[
            pl.BlockSpec((bc, lp), lambda i: (i, 0)),
            pl.BlockSpec((bc, lp), lambda i: (i, 0)),
            pl.BlockSpec((bc, lp), lambda i: (i, 0)),
        ],
        out_specs=pl.BlockSpec((1, 1), lambda i: (0, 0)),
        out_shape=jax.ShapeDtypeStruct((1, 1), jnp.float32),
    )(ids_p, m_p, vals_p)

    return (dense_tot[0, 0] - corr_tot[0, 0]) / (b * v)
